# baseline (device time: 23716 ns/iter reference)
import jax
import jax.numpy as jnp
from jax import lax
from jax.experimental import pallas as pl
from jax.experimental.pallas import tpu as pltpu

NX, NY, NZ = 2, 4, 4
NH = 4
MESH = pl.DeviceIdType.MESH


def kernel(x):
    _, m, n_total = x.shape
    ncol = n_total // NZ
    nrow = m // (NX * NY)
    half = NY * nrow
    nch = ncol // NH

    def body(x_ref, out_ref, xloc, own_buf, recv_p1, recv_p2, xr_own, xr_p2,
             local_sem, ss1, rs1, ss2, rs2, ss3a, rs3a, ss3b, rs3b):
        my_x = lax.axis_index("x")
        my_y = lax.axis_index("y")
        my_z = lax.axis_index("z")
        row_off = my_x * half + my_y * nrow
        xp = (1 - my_x, my_y, my_z)

        stripe = pltpu.make_async_copy(
            x_ref.at[0, pl.ds(row_off, nrow), :], xloc, local_sem)
        stripe.start()

        barrier_sem = pltpu.get_barrier_semaphore()
        for k in range(1, NZ):
            pl.semaphore_signal(
                barrier_sem, inc=1, device_id_type=MESH,
                device_id=(my_x, my_y, lax.rem(my_z + k, NZ)))
        for k in range(1, NY):
            pl.semaphore_signal(
                barrier_sem, inc=1, device_id_type=MESH,
                device_id=(my_x, lax.rem(my_y + k, NY), my_z))
        pl.semaphore_signal(barrier_sem, inc=1, device_id_type=MESH,
                            device_id=xp)
        pl.semaphore_wait(barrier_sem, NZ - 1 + NY - 1 + 1)
        stripe.wait()

        sends = []

        for h in range(NH):
            for k in range(1, NZ):
                d = lax.rem(my_z + k, NZ)
                rd = pltpu.make_async_remote_copy(
                    src_ref=xloc.at[:, pl.ds(d * ncol + h * nch, nch)],
                    dst_ref=recv_p1.at[h, NZ - 1 - k],
                    send_sem=ss1.at[h, k - 1],
                    recv_sem=rs1.at[h, NZ - 1 - k],
                    device_id=(my_x, my_y, d), device_id_type=MESH,
                )
                rd.start()
                sends.append(rd)

        rd3a = []
        for h in range(NH):
            for r in range(NZ - 1):
                pltpu.make_async_remote_copy(
                    src_ref=recv_p1.at[h, r], dst_ref=recv_p1.at[h, r],
                    send_sem=ss1.at[h, 0], recv_sem=rs1.at[h, r],
                    device_id=(my_x, my_y, my_z), device_id_type=MESH,
                ).wait_recv()
            own_buf[:, pl.ds(h * nch, nch)] = (
                xloc[:, pl.ds(my_z * ncol + h * nch, nch)]
                + recv_p1[h, 0] + recv_p1[h, 1] + recv_p1[h, 2]
            )
            for k in range(1, NY):
                p = lax.rem(my_y + k, NY)
                rd = pltpu.make_async_remote_copy(
                    src_ref=own_buf.at[:, pl.ds(h * nch, nch)],
                    dst_ref=recv_p2.at[NY - 1 - k, :, pl.ds(h * nch, nch)],
                    send_sem=ss2.at[h, k - 1],
                    recv_sem=rs2.at[h, NY - 1 - k],
                    device_id=(my_x, p, my_z), device_id_type=MESH,
                )
                rd.start()
                sends.append(rd)
            rd = pltpu.make_async_remote_copy(
                src_ref=own_buf.at[:, pl.ds(h * nch, nch)],
                dst_ref=xr_own.at[:, pl.ds(h * nch, nch)],
                send_sem=ss3a.at[h], recv_sem=rs3a.at[h],
                device_id=xp, device_id_type=MESH,
            )
            rd.start()
            rd3a.append(rd)
            sends.append(rd)

        rd3b = []
        for h in range(NH):
            for r in range(NY - 1):
                pltpu.make_async_remote_copy(
                    src_ref=recv_p2.at[r, :, pl.ds(h * nch, nch)],
                    dst_ref=recv_p2.at[r, :, pl.ds(h * nch, nch)],
                    send_sem=ss2.at[h, 0], recv_sem=rs2.at[h, r],
                    device_id=(my_x, my_y, my_z), device_id_type=MESH,
                ).wait_recv()
            rd = pltpu.make_async_remote_copy(
                src_ref=recv_p2.at[:, :, pl.ds(h * nch, nch)],
                dst_ref=xr_p2.at[:, :, pl.ds(h * nch, nch)],
                send_sem=ss3b.at[h], recv_sem=rs3b.at[h],
                device_id=xp, device_id_type=MESH,
            )
            rd.start()
            rd3b.append(rd)
            sends.append(rd)
            out_ref[pl.ds(row_off, nrow), pl.ds(h * nch, nch)] = (
                own_buf[:, pl.ds(h * nch, nch)])
            for r in range(NY - 1):
                y_src = lax.rem(my_y + 1 + r, NY)
                out_ref[pl.ds(my_x * half + y_src * nrow, nrow),
                        pl.ds(h * nch, nch)] = recv_p2[r, :, pl.ds(h * nch, nch)]

        other = (1 - my_x) * half
        for h in range(NH):
            rd3a[h].wait_recv()
            rd3b[h].wait_recv()
            out_ref[pl.ds(other + my_y * nrow, nrow), pl.ds(h * nch, nch)] = (
                xr_own[:, pl.ds(h * nch, nch)])
            for r in range(NY - 1):
                y_src = lax.rem(my_y + 1 + r, NY)
                out_ref[pl.ds(other + y_src * nrow, nrow),
                        pl.ds(h * nch, nch)] = xr_p2[r, :, pl.ds(h * nch, nch)]

        for rd in sends:
            rd.wait_send()

    return pl.pallas_call(
        body,
        out_shape=jax.ShapeDtypeStruct((m, ncol), jnp.float32),
        in_specs=[pl.BlockSpec(memory_space=pl.ANY)],
        out_specs=pl.BlockSpec(memory_space=pltpu.VMEM),
        scratch_shapes=[
            pltpu.VMEM((nrow, n_total), jnp.float32),
            pltpu.VMEM((nrow, ncol), jnp.float32),
            pltpu.VMEM((NH, NZ - 1, nrow, nch), jnp.float32),
            pltpu.VMEM((NY - 1, nrow, ncol), jnp.float32),
            pltpu.VMEM((nrow, ncol), jnp.float32),
            pltpu.VMEM((NY - 1, nrow, ncol), jnp.float32),
            pltpu.SemaphoreType.DMA,
            pltpu.SemaphoreType.DMA((NH, NZ - 1)),
            pltpu.SemaphoreType.DMA((NH, NZ - 1)),
            pltpu.SemaphoreType.DMA((NH, NY - 1)),
            pltpu.SemaphoreType.DMA((NH, NY - 1)),
            pltpu.SemaphoreType.DMA((NH,)),
            pltpu.SemaphoreType.DMA((NH,)),
            pltpu.SemaphoreType.DMA((NH,)),
            pltpu.SemaphoreType.DMA((NH,)),
        ],
        compiler_params=pltpu.CompilerParams(collective_id=0),
    )(x)
